# Initial kernel scaffold; baseline (speedup 1.0000x reference)
import jax
import jax.numpy as jnp
from jax.experimental import pallas as pl


def kernel(x_rna, edge_index_rna, e_rna, x_atac, edge_index_atac, e_atac, x_cell, W1_rna, b1_rna, W2_rna, b2_rna, Wu_rna, bu_rna, W1_atac, b1_atac, W2_atac, b2_atac, Wu_atac, bu_atac, Wc, bc):
    raise NotImplementedError("write your pallas kernel here")


# speedup vs baseline: 3.3509x; 3.3509x over previous
import functools

import jax
import jax.numpy as jnp
from jax import lax
from jax.experimental import pallas as pl
from jax.experimental.pallas import tpu as pltpu
from jax.experimental.pallas import tpu_sc as plsc

_N = 10000
_E = 320000
_D = 128
_DE = 16
_H = 128

_NC = 2
_NS = 16
_NW = _NC * _NS
_C = 80
_PER_W = _E // _NW
_NCH = _PER_W // _C
_NP = 10240
_RPS = _NP // _NS
_ZR = 128


def _pq_body(x_ref, wa_ref, wb_ref, p_ref, q_ref):
    x = x_ref[...]
    p_ref[...] = jnp.dot(x, wa_ref[...], preferred_element_type=jnp.float32)
    q_ref[...] = jnp.dot(x, wb_ref[...], preferred_element_type=jnp.float32)


def _node_tables(x, wa, wb):
    bs = 2000
    return pl.pallas_call(
        _pq_body,
        grid=(_N // bs,),
        in_specs=[
            pl.BlockSpec((bs, _D), lambda i: (i, 0)),
            pl.BlockSpec((_D, _H), lambda i: (0, 0)),
            pl.BlockSpec((_D, _H), lambda i: (0, 0)),
        ],
        out_specs=[
            pl.BlockSpec((bs, _H), lambda i: (i, 0)),
            pl.BlockSpec((bs, _H), lambda i: (i, 0)),
        ],
        out_shape=[
            jax.ShapeDtypeStruct((_N, _H), jnp.float32),
            jax.ShapeDtypeStruct((_N, _H), jnp.float32),
        ],
    )(x, wa, wb)


def _edge_body(g_ref, e_ref, w1c_ref, b1_ref, w2_ref, b2_ref, h3_ref):
    pre = g_ref[...] + jnp.dot(e_ref[...], w1c_ref[...],
                               preferred_element_type=jnp.float32) + b1_ref[...]
    h = jax.nn.silu(pre)
    h2 = jax.nn.silu(jnp.dot(h, w2_ref[...],
                             preferred_element_type=jnp.float32) + b2_ref[...])
    h3_ref[...] = jax.nn.silu(h2)


def _edge_mlp(g, e, w1c, b1, w2, b2):
    bs = 2000
    return pl.pallas_call(
        _edge_body,
        grid=(_E // bs,),
        in_specs=[
            pl.BlockSpec((bs, _H), lambda i: (i, 0)),
            pl.BlockSpec((bs, _DE), lambda i: (i, 0)),
            pl.BlockSpec((_DE, _H), lambda i: (0, 0)),
            pl.BlockSpec((1, _H), lambda i: (0, 0)),
            pl.BlockSpec((_H, _H), lambda i: (0, 0)),
            pl.BlockSpec((1, _H), lambda i: (0, 0)),
        ],
        out_specs=pl.BlockSpec((bs, _H), lambda i: (i, 0)),
        out_shape=jax.ShapeDtypeStruct((_E, _H), jnp.float32),
    )(g, e, w1c, b1.reshape(1, _H), w2, b2.reshape(1, _H))


def _final_body(agg_ref, wu_ref, bu_ref, out_ref):
    a = agg_ref[0] + agg_ref[1]
    out_ref[...] = jnp.dot(a, wu_ref[...],
                           preferred_element_type=jnp.float32) + bu_ref[...]


def _final_update(agg, wu, bu):
    bs = 2000
    return pl.pallas_call(
        _final_body,
        grid=(_N // bs,),
        in_specs=[
            pl.BlockSpec((2, bs, _H), lambda i: (0, i, 0)),
            pl.BlockSpec((_H, _H), lambda i: (0, 0)),
            pl.BlockSpec((1, _H), lambda i: (0, 0)),
        ],
        out_specs=pl.BlockSpec((bs, _H), lambda i: (i, 0)),
        out_shape=jax.ShapeDtypeStruct((_N, _H), jnp.float32),
    )(agg, wu, bu.reshape(1, _H))


def _cell_body(x_ref, wc_ref, bc_ref, out_ref):
    out_ref[...] = jax.nn.silu(
        jnp.dot(x_ref[...], wc_ref[...], preferred_element_type=jnp.float32)
        + bc_ref[...])


def _cell_branch(x, wc, bc):
    bs = 2000
    return pl.pallas_call(
        _cell_body,
        grid=(_N // bs,),
        in_specs=[
            pl.BlockSpec((bs, _D), lambda i: (i, 0)),
            pl.BlockSpec((_D, _H), lambda i: (0, 0)),
            pl.BlockSpec((1, _H), lambda i: (0, 0)),
        ],
        out_specs=pl.BlockSpec((bs, _H), lambda i: (i, 0)),
        out_shape=jax.ShapeDtypeStruct((_N, _H), jnp.float32),
    )(x, wc, bc.reshape(1, _H))


_MESH = plsc.VectorSubcoreMesh(core_axis_name="c", subcore_axis_name="s")


def _sc_gather_body(p_hbm, q_hbm, dst_hbm, src_hbm, g_hbm,
                    idx_d, idx_s, prow, qrow, sem_p, sem_q):
    wid = lax.axis_index("s") * _NC + lax.axis_index("c")
    base = wid * _PER_W

    def chunk(i, carry):
        off = base + i * _C
        pltpu.sync_copy(dst_hbm.at[pl.ds(off, _C)], idx_d)
        pltpu.sync_copy(src_hbm.at[pl.ds(off, _C)], idx_s)
        cp_p = pltpu.async_copy(p_hbm.at[idx_d], prow, sem_p)
        cp_q = pltpu.async_copy(q_hbm.at[idx_s], qrow, sem_q)
        cp_p.wait()
        cp_q.wait()

        def add_row(r, c2):
            for j in range(_H // 16):
                sl = pl.ds(j * 16, 16)
                prow[r, sl] = prow[r, sl] + qrow[r, sl]
            return c2

        lax.fori_loop(0, _C, add_row, 0)
        pltpu.sync_copy(prow, g_hbm.at[pl.ds(off, _C)])
        return carry

    lax.fori_loop(0, _NCH, chunk, 0)


@functools.partial(
    pl.kernel,
    mesh=_MESH,
    out_type=jax.ShapeDtypeStruct((_E, _H), jnp.float32),
    scratch_types=[
        pltpu.VMEM((_C,), jnp.int32),
        pltpu.VMEM((_C,), jnp.int32),
        pltpu.VMEM((_C, _H), jnp.float32),
        pltpu.VMEM((_C, _H), jnp.float32),
        pltpu.SemaphoreType.DMA,
        pltpu.SemaphoreType.DMA,
    ],
)
def _sc_gather(p_hbm, q_hbm, dst_hbm, src_hbm, g_hbm, *rest):
    _sc_gather_body(p_hbm, q_hbm, dst_hbm, src_hbm, g_hbm, *rest)


def _sc_scatter_body(h3_hbm, dst_hbm, out_hbm, idx_v, rows_v, zb, acc_sh):
    cid = lax.axis_index("c")
    sid = lax.axis_index("s")
    wid = sid * _NC + cid
    base = wid * _PER_W

    def zrow(r, c2):
        for j in range(_H // 16):
            zb[r, pl.ds(j * 16, 16)] = jnp.zeros((16,), jnp.float32)
        return c2

    lax.fori_loop(0, _ZR, zrow, 0)
    for k in range(_RPS // _ZR):
        pltpu.sync_copy(zb, acc_sh.at[pl.ds(sid * _RPS + k * _ZR, _ZR)])
    plsc.subcore_barrier()

    def chunk(i, carry):
        off = base + i * _C
        pltpu.sync_copy(dst_hbm.at[pl.ds(off, _C)], idx_v)
        pltpu.sync_copy(h3_hbm.at[pl.ds(off, _C)], rows_v)
        pltpu.sync_copy(rows_v, acc_sh.at[idx_v], add=True)
        return carry

    lax.fori_loop(0, _NCH, chunk, 0)
    plsc.subcore_barrier()

    pltpu.sync_copy(acc_sh.at[pl.ds(sid * _RPS, _RPS)],
                    out_hbm.at[cid, pl.ds(sid * _RPS, _RPS)])


@functools.partial(
    pl.kernel,
    mesh=_MESH,
    out_type=jax.ShapeDtypeStruct((_NC, _NP, _H), jnp.float32),
    scratch_types=[
        pltpu.VMEM((_C,), jnp.int32),
        pltpu.VMEM((_C, _H), jnp.float32),
        pltpu.VMEM((_ZR, _H), jnp.float32),
        pltpu.VMEM_SHARED((_NP, _H), jnp.float32),
    ],
)
def _sc_scatter(h3_hbm, dst_hbm, out_hbm, *rest):
    _sc_scatter_body(h3_hbm, dst_hbm, out_hbm, *rest)


def _modality(x, edge_index, e, w1, b1, w2, b2, wu, bu):
    src = edge_index[0].astype(jnp.int32)
    dst = edge_index[1].astype(jnp.int32)
    p, q = _node_tables(x, w1[:_D], w1[_D:2 * _D])
    g = _sc_gather(p, q, dst, src)
    h3 = _edge_mlp(g, e, w1[2 * _D:], b1, w2, b2)
    agg = _sc_scatter(h3, dst)
    return _final_update(agg, wu, bu)


def kernel(x_rna, edge_index_rna, e_rna, x_atac, edge_index_atac, e_atac,
           x_cell, W1_rna, b1_rna, W2_rna, b2_rna, Wu_rna, bu_rna,
           W1_atac, b1_atac, W2_atac, b2_atac, Wu_atac, bu_atac, Wc, bc):
    h_rna = _modality(x_rna, edge_index_rna, e_rna,
                      W1_rna, b1_rna, W2_rna, b2_rna, Wu_rna, bu_rna)
    h_atac = _modality(x_atac, edge_index_atac, e_atac,
                       W1_atac, b1_atac, W2_atac, b2_atac, Wu_atac, bu_atac)
    c = _cell_branch(x_cell, Wc, bc)
    return (h_rna, h_atac, c)


# speedup vs baseline: 4.7150x; 1.4071x over previous
import functools

import jax
import jax.numpy as jnp
from jax import lax
from jax.experimental import pallas as pl
from jax.experimental.pallas import tpu as pltpu
from jax.experimental.pallas import tpu_sc as plsc

_N = 10000
_E = 320000
_D = 128
_DE = 16
_H = 128

_NC = 2
_NS = 16
_NW = _NC * _NS
_C = 80
_PER_W = _E // _NW
_NCH = _PER_W // _C
_NP = 10240
_RPS = _NP // _NS
_ZR = 64


def _pq_body(x_ref, wa_ref, wb_ref, p_ref, q_ref):
    x = x_ref[...]
    p_ref[...] = jnp.dot(x, wa_ref[...], preferred_element_type=jnp.float32)
    q_ref[...] = jnp.dot(x, wb_ref[...], preferred_element_type=jnp.float32)


def _node_tables(x, wa, wb):
    bs = 2000
    return pl.pallas_call(
        _pq_body,
        grid=(_N // bs,),
        in_specs=[
            pl.BlockSpec((bs, _D), lambda i: (i, 0)),
            pl.BlockSpec((_D, _H), lambda i: (0, 0)),
            pl.BlockSpec((_D, _H), lambda i: (0, 0)),
        ],
        out_specs=[
            pl.BlockSpec((bs, _H), lambda i: (i, 0)),
            pl.BlockSpec((bs, _H), lambda i: (i, 0)),
        ],
        out_shape=[
            jax.ShapeDtypeStruct((_N, _H), jnp.float32),
            jax.ShapeDtypeStruct((_N, _H), jnp.float32),
        ],
    )(x, wa, wb)


def _edge_body(g_ref, e_ref, w1c_ref, b1_ref, w2_ref, b2_ref, h3_ref):
    pre = g_ref[...] + jnp.dot(e_ref[...], w1c_ref[...],
                               preferred_element_type=jnp.float32) + b1_ref[...]
    h = jax.nn.silu(pre)
    h2 = jax.nn.silu(jnp.dot(h, w2_ref[...],
                             preferred_element_type=jnp.float32) + b2_ref[...])
    h3_ref[...] = jax.nn.silu(h2)


def _edge_mlp(g, e, w1c, b1, w2, b2):
    bs = 2000
    return pl.pallas_call(
        _edge_body,
        grid=(_E // bs,),
        in_specs=[
            pl.BlockSpec((bs, _H), lambda i: (i, 0)),
            pl.BlockSpec((bs, _DE), lambda i: (i, 0)),
            pl.BlockSpec((_DE, _H), lambda i: (0, 0)),
            pl.BlockSpec((1, _H), lambda i: (0, 0)),
            pl.BlockSpec((_H, _H), lambda i: (0, 0)),
            pl.BlockSpec((1, _H), lambda i: (0, 0)),
        ],
        out_specs=pl.BlockSpec((bs, _H), lambda i: (i, 0)),
        out_shape=jax.ShapeDtypeStruct((_E, _H), jnp.float32),
    )(g, e, w1c, b1.reshape(1, _H), w2, b2.reshape(1, _H))


def _final_body(agg_ref, wu_ref, bu_ref, out_ref):
    a = agg_ref[0] + agg_ref[1]
    out_ref[...] = jnp.dot(a, wu_ref[...],
                           preferred_element_type=jnp.float32) + bu_ref[...]


def _final_update(agg, wu, bu):
    bs = 2000
    return pl.pallas_call(
        _final_body,
        grid=(_N // bs,),
        in_specs=[
            pl.BlockSpec((2, bs, _H), lambda i: (0, i, 0)),
            pl.BlockSpec((_H, _H), lambda i: (0, 0)),
            pl.BlockSpec((1, _H), lambda i: (0, 0)),
        ],
        out_specs=pl.BlockSpec((bs, _H), lambda i: (i, 0)),
        out_shape=jax.ShapeDtypeStruct((_N, _H), jnp.float32),
    )(agg, wu, bu.reshape(1, _H))


def _cell_body(x_ref, wc_ref, bc_ref, out_ref):
    out_ref[...] = jax.nn.silu(
        jnp.dot(x_ref[...], wc_ref[...], preferred_element_type=jnp.float32)
        + bc_ref[...])


def _cell_branch(x, wc, bc):
    bs = 2000
    return pl.pallas_call(
        _cell_body,
        grid=(_N // bs,),
        in_specs=[
            pl.BlockSpec((bs, _D), lambda i: (i, 0)),
            pl.BlockSpec((_D, _H), lambda i: (0, 0)),
            pl.BlockSpec((1, _H), lambda i: (0, 0)),
        ],
        out_specs=pl.BlockSpec((bs, _H), lambda i: (i, 0)),
        out_shape=jax.ShapeDtypeStruct((_N, _H), jnp.float32),
    )(x, wc, bc.reshape(1, _H))


_MESH = plsc.VectorSubcoreMesh(core_axis_name="c", subcore_axis_name="s")


def _sc_gather_body(p_hbm, q_hbm, dst3_hbm, src3_hbm, g_hbm,
                    idx_d, idx_s, prow0, qrow0, prow1, qrow1,
                    sp0, sq0, sp1, sq1):
    wid = lax.axis_index("s") * _NC + lax.axis_index("c")
    base = wid * _PER_W

    pltpu.sync_copy(dst3_hbm.at[wid], idx_d)
    pltpu.sync_copy(src3_hbm.at[wid], idx_s)

    bufs = ((prow0, qrow0, sp0, sq0), (prow1, qrow1, sp1, sq1))

    def start(c, b):
        prow, qrow, sp, sq = bufs[b]
        pltpu.async_copy(p_hbm.at[idx_d.at[c]], prow, sp)
        pltpu.async_copy(q_hbm.at[idx_s.at[c]], qrow, sq)

    def finish(c, b):
        prow, qrow, sp, sq = bufs[b]
        pltpu.make_async_copy(p_hbm.at[idx_d.at[c]], prow, sp).wait()
        pltpu.make_async_copy(q_hbm.at[idx_s.at[c]], qrow, sq).wait()

        def add_row(r, c2):
            for j in range(_H // 16):
                sl = pl.ds(j * 16, 16)
                prow[r, sl] = prow[r, sl] + qrow[r, sl]
            return c2

        lax.fori_loop(0, _C, add_row, 0)
        pltpu.sync_copy(prow, g_hbm.at[pl.ds(base + c * _C, _C)])

    start(0, 0)

    def pair(k, carry):
        c0 = 2 * k
        start(c0 + 1, 1)
        finish(c0, 0)
        start(c0 + 2, 0)
        finish(c0 + 1, 1)
        return carry

    lax.fori_loop(0, (_NCH - 1) // 2, pair, 0)
    finish(_NCH - 1, 0)


@functools.partial(
    pl.kernel,
    mesh=_MESH,
    out_type=jax.ShapeDtypeStruct((_E, _H), jnp.float32),
    scratch_types=[
        pltpu.VMEM((_NCH, _C), jnp.int32),
        pltpu.VMEM((_NCH, _C), jnp.int32),
        pltpu.VMEM((_C, _H), jnp.float32),
        pltpu.VMEM((_C, _H), jnp.float32),
        pltpu.VMEM((_C, _H), jnp.float32),
        pltpu.VMEM((_C, _H), jnp.float32),
        pltpu.SemaphoreType.DMA,
        pltpu.SemaphoreType.DMA,
        pltpu.SemaphoreType.DMA,
        pltpu.SemaphoreType.DMA,
    ],
)
def _sc_gather(p_hbm, q_hbm, dst3_hbm, src3_hbm, g_hbm, *rest):
    _sc_gather_body(p_hbm, q_hbm, dst3_hbm, src3_hbm, g_hbm, *rest)


def _sc_scatter_body(h3_hbm, dst3_hbm, out_hbm,
                     idx_v, rows0, rows1, zb, acc_sh, sr0, sr1):
    cid = lax.axis_index("c")
    sid = lax.axis_index("s")
    wid = sid * _NC + cid
    base = wid * _PER_W

    pltpu.sync_copy(dst3_hbm.at[wid], idx_v)

    def zrow(r, c2):
        for j in range(_H // 16):
            zb[r, pl.ds(j * 16, 16)] = jnp.zeros((16,), jnp.float32)
        return c2

    lax.fori_loop(0, _ZR, zrow, 0)
    for k in range(_RPS // _ZR):
        pltpu.sync_copy(zb, acc_sh.at[pl.ds(sid * _RPS + k * _ZR, _ZR)])
    plsc.subcore_barrier()

    bufs = ((rows0, sr0), (rows1, sr1))

    def start(c, b):
        rows, sr = bufs[b]
        pltpu.async_copy(h3_hbm.at[pl.ds(base + c * _C, _C)], rows, sr)

    def finish(c, b):
        rows, sr = bufs[b]
        pltpu.make_async_copy(
            h3_hbm.at[pl.ds(base + c * _C, _C)], rows, sr).wait()
        pltpu.sync_copy(rows, acc_sh.at[idx_v.at[c]], add=True)

    start(0, 0)

    def pair(k, carry):
        c0 = 2 * k
        start(c0 + 1, 1)
        finish(c0, 0)
        start(c0 + 2, 0)
        finish(c0 + 1, 1)
        return carry

    lax.fori_loop(0, (_NCH - 1) // 2, pair, 0)
    finish(_NCH - 1, 0)
    plsc.subcore_barrier()

    pltpu.sync_copy(acc_sh.at[pl.ds(sid * _RPS, _RPS)],
                    out_hbm.at[cid, pl.ds(sid * _RPS, _RPS)])


@functools.partial(
    pl.kernel,
    mesh=_MESH,
    out_type=jax.ShapeDtypeStruct((_NC, _NP, _H), jnp.float32),
    scratch_types=[
        pltpu.VMEM((_NCH, _C), jnp.int32),
        pltpu.VMEM((_C, _H), jnp.float32),
        pltpu.VMEM((_C, _H), jnp.float32),
        pltpu.VMEM((_ZR, _H), jnp.float32),
        pltpu.VMEM_SHARED((_NP, _H), jnp.float32),
        pltpu.SemaphoreType.DMA,
        pltpu.SemaphoreType.DMA,
    ],
)
def _sc_scatter(h3_hbm, dst3_hbm, out_hbm, *rest):
    _sc_scatter_body(h3_hbm, dst3_hbm, out_hbm, *rest)


def _modality(x, edge_index, e, w1, b1, w2, b2, wu, bu):
    src3 = edge_index[0].astype(jnp.int32).reshape(_NW, _NCH, _C)
    dst3 = edge_index[1].astype(jnp.int32).reshape(_NW, _NCH, _C)
    p, q = _node_tables(x, w1[:_D], w1[_D:2 * _D])
    g = _sc_gather(p, q, dst3, src3)
    h3 = _edge_mlp(g, e, w1[2 * _D:], b1, w2, b2)
    agg = _sc_scatter(h3, dst3)
    return _final_update(agg, wu, bu)


def kernel(x_rna, edge_index_rna, e_rna, x_atac, edge_index_atac, e_atac,
           x_cell, W1_rna, b1_rna, W2_rna, b2_rna, Wu_rna, bu_rna,
           W1_atac, b1_atac, W2_atac, b2_atac, Wu_atac, bu_atac, Wc, bc):
    h_rna = _modality(x_rna, edge_index_rna, e_rna,
                      W1_rna, b1_rna, W2_rna, b2_rna, Wu_rna, bu_rna)
    h_atac = _modality(x_atac, edge_index_atac, e_atac,
                       W1_atac, b1_atac, W2_atac, b2_atac, Wu_atac, bu_atac)
    c = _cell_branch(x_cell, Wc, bc)
    return (h_rna, h_atac, c)


# speedup vs baseline: 5.0485x; 1.0707x over previous
import functools

import jax
import jax.numpy as jnp
from jax import lax
from jax.experimental import pallas as pl
from jax.experimental.pallas import tpu as pltpu
from jax.experimental.pallas import tpu_sc as plsc

_N = 10000
_E = 320000
_D = 128
_DE = 16
_H = 128

_NC = 2
_NS = 16
_NW = _NC * _NS
_C = 80
_PER_W = _E // _NW
_NCH = _PER_W // _C
_NP = 10240
_RPS = _NP // _NS
_ZR = 64


def _pq_body(x_ref, wa_ref, wb_ref, p_ref, q_ref):
    x = x_ref[...]
    p_ref[...] = jnp.dot(x, wa_ref[...], preferred_element_type=jnp.float32)
    q_ref[...] = jnp.dot(x, wb_ref[...], preferred_element_type=jnp.float32)


def _node_tables(x, wa, wb):
    bs = 2000
    return pl.pallas_call(
        _pq_body,
        grid=(_N // bs,),
        in_specs=[
            pl.BlockSpec((bs, _D), lambda i: (i, 0)),
            pl.BlockSpec((_D, _H), lambda i: (0, 0)),
            pl.BlockSpec((_D, _H), lambda i: (0, 0)),
        ],
        out_specs=[
            pl.BlockSpec((bs, _H), lambda i: (i, 0)),
            pl.BlockSpec((bs, _H), lambda i: (i, 0)),
        ],
        out_shape=[
            jax.ShapeDtypeStruct((_N, _H), jnp.float32),
            jax.ShapeDtypeStruct((_N, _H), jnp.float32),
        ],
    )(x, wa, wb)


def _edge_body(g_ref, e_ref, w1c_ref, b1_ref, w2_ref, b2_ref, h3_ref):
    pre = g_ref[...] + jnp.dot(e_ref[...], w1c_ref[...],
                               preferred_element_type=jnp.float32) + b1_ref[...]
    h = jax.nn.silu(pre)
    h2 = jax.nn.silu(jnp.dot(h, w2_ref[...],
                             preferred_element_type=jnp.float32) + b2_ref[...])
    h3_ref[...] = jax.nn.silu(h2)


def _edge_mlp(g, e, w1c, b1, w2, b2):
    bs = 4000
    return pl.pallas_call(
        _edge_body,
        grid=(_E // bs,),
        in_specs=[
            pl.BlockSpec((bs, _H), lambda i: (i, 0)),
            pl.BlockSpec((bs, _DE), lambda i: (i, 0)),
            pl.BlockSpec((_DE, _H), lambda i: (0, 0)),
            pl.BlockSpec((1, _H), lambda i: (0, 0)),
            pl.BlockSpec((_H, _H), lambda i: (0, 0)),
            pl.BlockSpec((1, _H), lambda i: (0, 0)),
        ],
        out_specs=pl.BlockSpec((bs, _H), lambda i: (i, 0)),
        out_shape=jax.ShapeDtypeStruct((_E, _H), jnp.float32),
    )(g, e, w1c, b1.reshape(1, _H), w2, b2.reshape(1, _H))


def _final_body(agg_ref, wu_ref, bu_ref, out_ref):
    a = agg_ref[0] + agg_ref[1]
    out_ref[...] = jnp.dot(a, wu_ref[...],
                           preferred_element_type=jnp.float32) + bu_ref[...]


def _final_update(agg, wu, bu):
    bs = 2000
    return pl.pallas_call(
        _final_body,
        grid=(_N // bs,),
        in_specs=[
            pl.BlockSpec((2, bs, _H), lambda i: (0, i, 0)),
            pl.BlockSpec((_H, _H), lambda i: (0, 0)),
            pl.BlockSpec((1, _H), lambda i: (0, 0)),
        ],
        out_specs=pl.BlockSpec((bs, _H), lambda i: (i, 0)),
        out_shape=jax.ShapeDtypeStruct((_N, _H), jnp.float32),
    )(agg, wu, bu.reshape(1, _H))


def _cell_body(x_ref, wc_ref, bc_ref, out_ref):
    out_ref[...] = jax.nn.silu(
        jnp.dot(x_ref[...], wc_ref[...], preferred_element_type=jnp.float32)
        + bc_ref[...])


def _cell_branch(x, wc, bc):
    bs = 2000
    return pl.pallas_call(
        _cell_body,
        grid=(_N // bs,),
        in_specs=[
            pl.BlockSpec((bs, _D), lambda i: (i, 0)),
            pl.BlockSpec((_D, _H), lambda i: (0, 0)),
            pl.BlockSpec((1, _H), lambda i: (0, 0)),
        ],
        out_specs=pl.BlockSpec((bs, _H), lambda i: (i, 0)),
        out_shape=jax.ShapeDtypeStruct((_N, _H), jnp.float32),
    )(x, wc, bc.reshape(1, _H))


_MESH = plsc.VectorSubcoreMesh(core_axis_name="c", subcore_axis_name="s")


def _sc_gather_body(p_hbm, q_hbm, dst3_hbm, src3_hbm, g_hbm,
                    idx_d, idx_s, prow0, qrow0, prow1, qrow1,
                    sp0, sq0, sp1, sq1):
    wid = lax.axis_index("s") * _NC + lax.axis_index("c")
    base = wid * _PER_W

    pltpu.sync_copy(dst3_hbm.at[wid], idx_d)
    pltpu.sync_copy(src3_hbm.at[wid], idx_s)

    bufs = ((prow0, qrow0, sp0, sq0), (prow1, qrow1, sp1, sq1))

    def start(c, b, nch):
        prow, qrow, sp, sq = bufs[b]
        for j in range(nch):
            sub = pl.ds(j * _C, _C)
            pltpu.async_copy(p_hbm.at[idx_d.at[c + j]], prow.at[sub], sp)
            pltpu.async_copy(q_hbm.at[idx_s.at[c + j]], qrow.at[sub], sq)

    def finish(c, b, nch):
        prow, qrow, sp, sq = bufs[b]
        for j in range(nch):
            sub = pl.ds(j * _C, _C)
            pltpu.make_async_copy(
                p_hbm.at[idx_d.at[c + j]], prow.at[sub], sp).wait()
            pltpu.make_async_copy(
                q_hbm.at[idx_s.at[c + j]], qrow.at[sub], sq).wait()

        def add_row(r, c2):
            for j in range(_H // 16):
                sl = pl.ds(j * 16, 16)
                prow[r, sl] = prow[r, sl] + qrow[r, sl]
            return c2

        lax.fori_loop(0, nch * _C, add_row, 0)
        pltpu.sync_copy(prow.at[pl.ds(0, nch * _C)],
                        g_hbm.at[pl.ds(base + c * _C, nch * _C)])

    start(0, 0, 2)

    def spair(k, carry):
        c = 4 * k
        start(c + 2, 1, 2)
        finish(c, 0, 2)
        start(c + 4, 0, 2)
        finish(c + 2, 1, 2)
        return carry

    lax.fori_loop(0, (_NCH - 5) // 4, spair, 0)
    start(_NCH - 3, 1, 2)
    finish(_NCH - 5, 0, 2)
    start(_NCH - 1, 0, 1)
    finish(_NCH - 3, 1, 2)
    finish(_NCH - 1, 0, 1)


@functools.partial(
    pl.kernel,
    mesh=_MESH,
    out_type=jax.ShapeDtypeStruct((_E, _H), jnp.float32),
    scratch_types=[
        pltpu.VMEM((_NCH, _C), jnp.int32),
        pltpu.VMEM((_NCH, _C), jnp.int32),
        pltpu.VMEM((2 * _C, _H), jnp.float32),
        pltpu.VMEM((2 * _C, _H), jnp.float32),
        pltpu.VMEM((2 * _C, _H), jnp.float32),
        pltpu.VMEM((2 * _C, _H), jnp.float32),
        pltpu.SemaphoreType.DMA,
        pltpu.SemaphoreType.DMA,
        pltpu.SemaphoreType.DMA,
        pltpu.SemaphoreType.DMA,
    ],
)
def _sc_gather(p_hbm, q_hbm, dst3_hbm, src3_hbm, g_hbm, *rest):
    _sc_gather_body(p_hbm, q_hbm, dst3_hbm, src3_hbm, g_hbm, *rest)


def _sc_scatter_body(h3_hbm, dst3_hbm, out_hbm,
                     idx_v, rows0, rows1, zb, acc_sh, sr0, sr1):
    cid = lax.axis_index("c")
    sid = lax.axis_index("s")
    wid = sid * _NC + cid
    base = wid * _PER_W

    pltpu.sync_copy(dst3_hbm.at[wid], idx_v)

    def zrow(r, c2):
        for j in range(_H // 16):
            zb[r, pl.ds(j * 16, 16)] = jnp.zeros((16,), jnp.float32)
        return c2

    lax.fori_loop(0, _ZR, zrow, 0)
    for k in range(_RPS // _ZR):
        pltpu.sync_copy(zb, acc_sh.at[pl.ds(sid * _RPS + k * _ZR, _ZR)])
    plsc.subcore_barrier()

    bufs = ((rows0, sr0), (rows1, sr1))

    def start(c, b):
        rows, sr = bufs[b]
        pltpu.async_copy(h3_hbm.at[pl.ds(base + c * _C, _C)], rows, sr)

    def finish(c, b):
        rows, sr = bufs[b]
        pltpu.make_async_copy(
            h3_hbm.at[pl.ds(base + c * _C, _C)], rows, sr).wait()
        pltpu.sync_copy(rows, acc_sh.at[idx_v.at[c]], add=True)

    start(0, 0)

    def pair(k, carry):
        c0 = 2 * k
        start(c0 + 1, 1)
        finish(c0, 0)
        start(c0 + 2, 0)
        finish(c0 + 1, 1)
        return carry

    lax.fori_loop(0, (_NCH - 1) // 2, pair, 0)
    finish(_NCH - 1, 0)
    plsc.subcore_barrier()

    pltpu.sync_copy(acc_sh.at[pl.ds(sid * _RPS, _RPS)],
                    out_hbm.at[cid, pl.ds(sid * _RPS, _RPS)])


@functools.partial(
    pl.kernel,
    mesh=_MESH,
    out_type=jax.ShapeDtypeStruct((_NC, _NP, _H), jnp.float32),
    scratch_types=[
        pltpu.VMEM((_NCH, _C), jnp.int32),
        pltpu.VMEM((_C, _H), jnp.float32),
        pltpu.VMEM((_C, _H), jnp.float32),
        pltpu.VMEM((_ZR, _H), jnp.float32),
        pltpu.VMEM_SHARED((_NP, _H), jnp.float32),
        pltpu.SemaphoreType.DMA,
        pltpu.SemaphoreType.DMA,
    ],
)
def _sc_scatter(h3_hbm, dst3_hbm, out_hbm, *rest):
    _sc_scatter_body(h3_hbm, dst3_hbm, out_hbm, *rest)


def _modality(x, edge_index, e, w1, b1, w2, b2, wu, bu):
    src3 = edge_index[0].astype(jnp.int32).reshape(_NW, _NCH, _C)
    dst3 = edge_index[1].astype(jnp.int32).reshape(_NW, _NCH, _C)
    p, q = _node_tables(x, w1[:_D], w1[_D:2 * _D])
    g = _sc_gather(p, q, dst3, src3)
    h3 = _edge_mlp(g, e, w1[2 * _D:], b1, w2, b2)
    agg = _sc_scatter(h3, dst3)
    return _final_update(agg, wu, bu)


def kernel(x_rna, edge_index_rna, e_rna, x_atac, edge_index_atac, e_atac,
           x_cell, W1_rna, b1_rna, W2_rna, b2_rna, Wu_rna, bu_rna,
           W1_atac, b1_atac, W2_atac, b2_atac, Wu_atac, bu_atac, Wc, bc):
    h_rna = _modality(x_rna, edge_index_rna, e_rna,
                      W1_rna, b1_rna, W2_rna, b2_rna, Wu_rna, bu_rna)
    h_atac = _modality(x_atac, edge_index_atac, e_atac,
                       W1_atac, b1_atac, W2_atac, b2_atac, Wu_atac, bu_atac)
    c = _cell_branch(x_cell, Wc, bc)
    return (h_rna, h_atac, c)
